# baseline (speedup 1.0000x reference)
import jax
import jax.numpy as jnp
from jax import lax
from jax.experimental import pallas as pl
from jax.experimental.pallas import tpu as pltpu
from jax.experimental.pallas import tpu_sc as plsc

N_NODES = 10000
N_EDGES = 320000
D = 128

NC = 2
NS = 16
NW = NC * NS
EPW = N_EDGES // NW
CHUNK = 80
NCHUNK = EPW // CHUNK
IBLK = 25
NIB = NCHUNK // IBLK
NPAD_A = 10112
RPT = NPAD_A // NS
NPAD_H = 10240
HROWS = NPAD_H // D


def _sc_body(x_hbm, ei_hbm, acc_out, deg_out,
             src_v, dst_v, rows_a, rows_b, rows_c, hist_v, iota_v,
             acc_sh, deg_sh, sem_a, sem_b, sem_c, sem_sa, sem_sb, sem_sc):
    cid = lax.axis_index("c")
    sid = lax.axis_index("s")
    wid = sid * NC + cid

    z16 = jnp.zeros((16,), jnp.float32)

    def zhist(i, _):
        for c in range(D // 16):
            hist_v[i, pl.ds(c * 16, 16)] = z16
        return 0

    lax.fori_loop(0, HROWS, zhist, 0)

    def zrow(i, _):
        for c in range(D // 16):
            rows_a[i, pl.ds(c * 16, 16)] = z16
        return 0

    lax.fori_loop(0, CHUNK, zrow, 0)

    for k in range(RPT // CHUNK):
        pltpu.sync_copy(
            rows_a, acc_sh.at[pl.ds(sid * RPT + k * CHUNK, CHUNK)])
    pltpu.sync_copy(rows_a.at[pl.ds(0, RPT % CHUNK)],
                    acc_sh.at[pl.ds(sid * RPT + RPT - RPT % CHUNK,
                                    RPT % CHUNK)])

    @pl.when(sid == 0)
    def _():
        pltpu.sync_copy(rows_a, deg_sh)

    for k in range(HROWS // 16):
        iota_v[0, pl.ds(k * 16, 16)] = lax.iota(jnp.int32, 16) + (k * 16)

    plsc.subcore_barrier()

    ones16 = jnp.full((16,), 1.0, jnp.float32)
    bufs = (rows_a, rows_b, rows_c)
    gsems = (sem_a, sem_b, sem_c)
    ssems = (sem_sa, sem_sb, sem_sc)

    def block(ib, _):
        pltpu.sync_copy(ei_hbm.at[0, wid, ib], src_v)
        pltpu.sync_copy(ei_hbm.at[1, wid, ib], dst_v)
        pltpu.async_copy(x_hbm.at[src_v.at[0]], rows_a, sem_a)
        pltpu.async_copy(x_hbm.at[src_v.at[1]], rows_b, sem_b)

        def step(j, _):
            @pl.when(j + 1 < IBLK)
            def _():
                for b in range(3):
                    @pl.when(lax.rem(j + 1, 3) == b)
                    def _(b=b):
                        @pl.when(j >= 2)
                        def _(b=b):
                            pltpu.make_async_copy(
                                bufs[b], acc_sh.at[dst_v.at[j]],
                                ssems[b]).wait()
                        pltpu.async_copy(
                            x_hbm.at[src_v.at[j + 1]], bufs[b], gsems[b])

            for k in range(CHUNK // 16):
                idx = dst_v[j, pl.ds(k * 16, 16)]
                r = lax.shift_right_logical(idx, 7)
                c = lax.bitwise_and(idx, 127)
                plsc.addupdate_scatter(hist_v, [r, c], ones16)

            for b in range(3):
                @pl.when(lax.rem(j, 3) == b)
                def _(b=b):
                    pltpu.make_async_copy(
                        x_hbm.at[src_v.at[j]], bufs[b], gsems[b]).wait()
                    pltpu.async_copy(bufs[b], acc_sh.at[dst_v.at[j]],
                                     ssems[b], add=True)

            return 0

        lax.fori_loop(0, IBLK, step, 0)
        for b in range(3):
            pltpu.make_async_copy(
                bufs[b], acc_sh.at[dst_v.at[0]], ssems[b]).wait()
        return 0

    lax.fori_loop(0, NIB, block, 0)

    pltpu.sync_copy(hist_v, deg_sh.at[iota_v.at[0]], add=True)
    plsc.subcore_barrier()

    pltpu.sync_copy(acc_sh.at[pl.ds(sid * RPT, RPT)],
                    acc_out.at[cid, pl.ds(sid * RPT, RPT)])

    @pl.when(sid < HROWS // 8)
    def _():
        pltpu.sync_copy(deg_sh.at[pl.ds(sid * 8, 8)],
                        deg_out.at[cid, pl.ds(sid * 8, 8)])


def _tc_body(p_ref, d_ref, x_ref, w_ref, b_ref, o_ref):
    p = p_ref[0] + p_ref[1]
    dg = d_ref[0] + d_ref[1]
    agg = p / jnp.maximum(dg, 1.0)
    h = jnp.dot(agg, w_ref[...], preferred_element_type=jnp.float32) + b_ref[...]
    o_ref[...] = jnp.maximum(h, 0.0) + x_ref[...]


def kernel(x, edge_index, W, b):
    ei = edge_index.astype(jnp.int32).reshape(2, NW, NIB, IBLK, CHUNK)

    mesh = plsc.VectorSubcoreMesh(core_axis_name="c", subcore_axis_name="s")
    acc_p, deg_p = pl.kernel(
        _sc_body,
        out_type=(
            jax.ShapeDtypeStruct((NC, NPAD_A, D), jnp.float32),
            jax.ShapeDtypeStruct((NC, HROWS, D), jnp.float32),
        ),
        mesh=mesh,
        compiler_params=pltpu.CompilerParams(needs_layout_passes=False),
        scratch_types=[
            pltpu.VMEM((IBLK, CHUNK), jnp.int32),
            pltpu.VMEM((IBLK, CHUNK), jnp.int32),
            pltpu.VMEM((CHUNK, D), jnp.float32),
            pltpu.VMEM((CHUNK, D), jnp.float32),
            pltpu.VMEM((CHUNK, D), jnp.float32),
            pltpu.VMEM((HROWS, D), jnp.float32),
            pltpu.VMEM((1, HROWS), jnp.int32),
            pltpu.VMEM_SHARED((NPAD_A, D), jnp.float32),
            pltpu.VMEM_SHARED((HROWS, D), jnp.float32),
            pltpu.SemaphoreType.DMA,
            pltpu.SemaphoreType.DMA,
            pltpu.SemaphoreType.DMA,
            pltpu.SemaphoreType.DMA,
            pltpu.SemaphoreType.DMA,
            pltpu.SemaphoreType.DMA,
        ],
    )(x, ei)

    deg_flat = deg_p.reshape(NC, NPAD_H, 1)

    R = 1000
    grid = (N_NODES // R,)
    h = pl.pallas_call(
        _tc_body,
        grid=grid,
        in_specs=[
            pl.BlockSpec((NC, R, D), lambda i: (0, i, 0)),
            pl.BlockSpec((NC, R, 1), lambda i: (0, i, 0)),
            pl.BlockSpec((R, D), lambda i: (i, 0)),
            pl.BlockSpec((D, D), lambda i: (0, 0)),
            pl.BlockSpec((1, D), lambda i: (0, 0)),
        ],
        out_specs=pl.BlockSpec((R, D), lambda i: (i, 0)),
        out_shape=jax.ShapeDtypeStruct((N_NODES, D), jnp.float32),
    )(acc_p, deg_flat, x, W, b.reshape(1, D))
    return h
